# Initial kernel scaffold; baseline (speedup 1.0000x reference)
import functools

import jax
import jax.numpy as jnp
from jax import lax
from jax.experimental import pallas as pl
from jax.experimental.pallas import tpu as pltpu
from jax.experimental.pallas import tpu_sc as plsc

_D = 64
_B = 4096 * 200
_NC, _NS = 2, 16
_NW = _NC * _NS
_BPW = _B // _NW
_C = 512
_G = _C // 128
_NCH = _BPW // _C


def _sc_body(p_idx, h_idx, table, p_out, h_out, idx_v, rows_v, gsem):
    wid = lax.axis_index("s") * _NC + lax.axis_index("c")
    for src, dst in ((p_idx, p_out), (h_idx, h_out)):

        @pl.loop(0, _NCH)
        def _chunk(c):
            base = wid * _BPW + c * _C
            rbase = wid * (_BPW // 128) + c * _G
            pltpu.sync_copy(src.at[pl.ds(rbase, _G)], idx_v)
            copies = [
                pltpu.async_copy(table.at[idx_v.at[j]],
                                 rows_v.at[pl.ds(j * 128, 128)], gsem)
                for j in range(_G)
            ]
            for cp in copies:
                cp.wait()
            pltpu.sync_copy(rows_v, dst.at[pl.ds(base, _C)])


@functools.partial(
    pl.kernel,
    out_type=(jax.ShapeDtypeStruct((_B, _D), jnp.float32),
              jax.ShapeDtypeStruct((_B, _D), jnp.float32)),
    mesh=plsc.VectorSubcoreMesh(core_axis_name="c", subcore_axis_name="s"),
    scratch_types=[
        pltpu.VMEM((_G, 128), jnp.int32),
        pltpu.VMEM((_C, _D), jnp.float32),
        pltpu.SemaphoreType.DMA,
    ],
)
def _embed_lookup(p_idx, h_idx, table, p_out, h_out, idx_v, rows_v, gsem):
    _sc_body(p_idx, h_idx, table, p_out, h_out, idx_v, rows_v, gsem)


def kernel(premises, hypotheses, labels, table):
    p = premises.reshape(_B // 128, 128)
    h = hypotheses.reshape(_B // 128, 128)
    p_emb, h_emb = _embed_lookup(p, h, table)
    return (p_emb.reshape(4096, 200, _D),
            h_emb.reshape(4096, 200, _D),
            labels)


# speedup vs baseline: 4.2712x; 4.2712x over previous
import functools

import jax
import jax.numpy as jnp
from jax import lax
from jax.experimental import pallas as pl
from jax.experimental.pallas import tpu as pltpu
from jax.experimental.pallas import tpu_sc as plsc

_D = 64
_B = 4096 * 200
_NC, _NS = 2, 16
_NW = _NC * _NS
_BPW = _B // _NW
_C = 512
_G = _C // 128
_NCH = _BPW // _C


def _sc_body(p_idx, h_idx, table, p_out, h_out, idx_v, rows_v, gsem):
    wid = lax.axis_index("s") * _NC + lax.axis_index("c")
    for src, dst in ((p_idx, p_out), (h_idx, h_out)):

        @pl.loop(0, _NCH)
        def _chunk(c):
            base = wid * _BPW + c * _C
            rbase = wid * (_BPW // 128) + c * _G
            pltpu.sync_copy(src.at[pl.ds(rbase, _G)], idx_v)
            copies = [
                pltpu.async_copy(table.at[idx_v.at[j]],
                                 rows_v.at[pl.ds(j * 128, 128)], gsem)
                for j in range(_G)
            ]
            for cp in copies:
                cp.wait()
            pltpu.sync_copy(rows_v, dst.at[pl.ds(base, _C)])


@functools.partial(
    pl.kernel,
    out_type=(jax.ShapeDtypeStruct((_B, _D), jnp.float32),
              jax.ShapeDtypeStruct((_B, _D), jnp.float32)),
    mesh=plsc.VectorSubcoreMesh(core_axis_name="c", subcore_axis_name="s"),
    compiler_params=pltpu.CompilerParams(use_tc_tiling_on_sc=False),
    scratch_types=[
        pltpu.VMEM((_G, 128), jnp.int32),
        pltpu.VMEM((_C, _D), jnp.float32),
        pltpu.SemaphoreType.DMA,
    ],
)
def _embed_lookup(p_idx, h_idx, table, p_out, h_out, idx_v, rows_v, gsem):
    _sc_body(p_idx, h_idx, table, p_out, h_out, idx_v, rows_v, gsem)


def kernel(premises, hypotheses, labels, table):
    p = premises.reshape(_B // 128, 128)
    h = hypotheses.reshape(_B // 128, 128)
    p_emb, h_emb = _embed_lookup(p, h, table)
    return (p_emb.reshape(4096, 200, _D),
            h_emb.reshape(4096, 200, _D),
            labels)


# speedup vs baseline: 4.6484x; 1.0883x over previous
import functools

import jax
import jax.numpy as jnp
from jax import lax
from jax.experimental import pallas as pl
from jax.experimental.pallas import tpu as pltpu
from jax.experimental.pallas import tpu_sc as plsc

_D = 64
_B = 4096 * 200
_NC, _NS = 2, 16
_NW = _NC * _NS
_BPW = _B // _NW
_R = _BPW // 128
_C = 512
_G = _C // 128
_NCH = _BPW // _C


def _fire_gathers(table, idx_v, row0, rows_b, gsem):
    return [
        pltpu.async_copy(table.at[idx_v.at[row0 + j]],
                         rows_b.at[pl.ds(j * 128, 128)], gsem)
        for j in range(_G)
    ]


def _sc_body(p_idx, h_idx, table, p_out, h_out,
             pidx_v, hidx_v, rows0, rows1, gsem0, gsem1, osem0, osem1):
    wid = lax.axis_index("s") * _NC + lax.axis_index("c")
    pltpu.sync_copy(p_idx.at[pl.ds(wid * _R, _R)], pidx_v)
    pltpu.sync_copy(h_idx.at[pl.ds(wid * _R, _R)], hidx_v)

    rows = (rows0, rows1)
    gsems = (gsem0, gsem1)
    osems = (osem0, osem1)
    obase = wid * _BPW

    for idx_v, dst in ((pidx_v, p_out), (hidx_v, h_out)):
        for b in range(2):
            for cp in _fire_gathers(table, idx_v, b * _G, rows[b], gsems[b]):
                cp.wait()
            pltpu.async_copy(rows[b], dst.at[pl.ds(obase + b * _C, _C)],
                             osems[b])

        @pl.loop(2, _NCH, step=2)
        def _steady(c0):
            for b in range(2):
                c = c0 + b
                pltpu.make_async_copy(
                    rows[b], dst.at[pl.ds(obase, _C)], osems[b]).wait()
                for cp in _fire_gathers(table, idx_v, c * _G, rows[b],
                                        gsems[b]):
                    cp.wait()
                pltpu.async_copy(rows[b], dst.at[pl.ds(obase + c * _C, _C)],
                                 osems[b])

        for b in range(2):
            pltpu.make_async_copy(
                rows[b], dst.at[pl.ds(obase, _C)], osems[b]).wait()


@functools.partial(
    pl.kernel,
    out_type=(jax.ShapeDtypeStruct((_B, _D), jnp.float32),
              jax.ShapeDtypeStruct((_B, _D), jnp.float32)),
    mesh=plsc.VectorSubcoreMesh(core_axis_name="c", subcore_axis_name="s"),
    compiler_params=pltpu.CompilerParams(use_tc_tiling_on_sc=False),
    scratch_types=[
        pltpu.VMEM((_R, 128), jnp.int32),
        pltpu.VMEM((_R, 128), jnp.int32),
        pltpu.VMEM((_C, _D), jnp.float32),
        pltpu.VMEM((_C, _D), jnp.float32),
        pltpu.SemaphoreType.DMA,
        pltpu.SemaphoreType.DMA,
        pltpu.SemaphoreType.DMA,
        pltpu.SemaphoreType.DMA,
    ],
)
def _embed_lookup(p_idx, h_idx, table, p_out, h_out,
                  pidx_v, hidx_v, rows0, rows1, gsem0, gsem1, osem0, osem1):
    _sc_body(p_idx, h_idx, table, p_out, h_out,
             pidx_v, hidx_v, rows0, rows1, gsem0, gsem1, osem0, osem1)


def kernel(premises, hypotheses, labels, table):
    p = premises.reshape(_B // 128, 128)
    h = hypotheses.reshape(_B // 128, 128)
    p_emb, h_emb = _embed_lookup(p, h, table)
    return (p_emb.reshape(4096, 200, _D),
            h_emb.reshape(4096, 200, _D),
            labels)
